# Initial kernel scaffold; baseline (speedup 1.0000x reference)
import jax
import jax.numpy as jnp
from jax.experimental import pallas as pl


def kernel(x, Wr, Wg, Wu, Wd):
    raise NotImplementedError("write your pallas kernel here")


# speedup vs baseline: 1.0259x; 1.0259x over previous
import functools

import jax
import jax.numpy as jnp
from jax import lax
from jax.experimental import pallas as pl
from jax.experimental.pallas import tpu as pltpu
from jax.experimental.pallas import tpu_sc as plsc

T = 4096
D = 1024
F = 512
E = 16
K = 2
BLK = 256
CAP = T * K + E * BLK
NBLK = CAP // BLK
NTILES = 32
ROWS_PER_TILE = CAP // NTILES
GCHUNK = 96
ENTRIES = T * K


def _router_body(x_ref, wr_ref, pos_ref, gate_ref, blke_ref, blkv_ref):
    x = x_ref[...]
    wr = wr_ref[...]
    logits = lax.dot_general(x, wr, (((1,), (1,)), ((), ())),
                             preferred_element_type=jnp.float32,
                             precision=lax.Precision.HIGHEST)
    m = jnp.max(logits, axis=1, keepdims=True)
    p = jnp.exp(logits - m)
    p = p / jnp.sum(p, axis=1, keepdims=True)

    lane = lax.broadcasted_iota(jnp.int32, (T, E), 1)
    m1 = jnp.max(p, axis=1, keepdims=True)
    i1 = jnp.min(jnp.where(p == m1, lane, E), axis=1, keepdims=True)
    p2 = jnp.where(lane == i1, -1.0, p)
    m2 = jnp.max(p2, axis=1, keepdims=True)
    i2 = jnp.min(jnp.where(p2 == m2, lane, E), axis=1, keepdims=True)
    s = m1 + m2
    g1 = m1 / s
    g2 = m2 / s

    oh1 = (lane == i1)
    oh2 = (lane == i2)
    ohc = jnp.where(oh1 | oh2, 1.0, 0.0)

    c = ohc
    k = 1
    while k < T:
        shifted = jnp.concatenate(
            [jnp.zeros((k, E), jnp.float32), c[: T - k, :]], axis=0)
        c = c + shifted
        k *= 2
    c_excl = c - ohc
    r1 = jnp.sum(jnp.where(oh1, c_excl, 0.0), axis=1, keepdims=True)
    r2 = jnp.sum(jnp.where(oh2, c_excl, 0.0), axis=1, keepdims=True)

    counts = c[T - 1 : T, :]
    padded = jnp.floor((counts + (BLK - 1)) / BLK) * BLK
    ri = lax.broadcasted_iota(jnp.int32, (E, E), 0)
    ci = lax.broadcasted_iota(jnp.int32, (E, E), 1)
    ut = jnp.where(ri <= ci, 1.0, 0.0)
    cum = lax.dot_general(padded, ut, (((1,), (0,)), ((), ())),
                          preferred_element_type=jnp.float32)
    offs = cum - padded

    offs_b = jnp.broadcast_to(offs, (T, E))
    o1 = jnp.sum(jnp.where(oh1, offs_b, 0.0), axis=1, keepdims=True)
    o2 = jnp.sum(jnp.where(oh2, offs_b, 0.0), axis=1, keepdims=True)
    pos1 = (o1 + r1).astype(jnp.int32)
    pos2 = (o2 + r2).astype(jnp.int32)
    pos_ref[...] = jnp.concatenate([pos1, pos2], axis=1)
    gate_ref[...] = jnp.concatenate([g1, g2], axis=1)

    nb = blke_ref.shape[0]
    sb = lax.broadcasted_iota(jnp.int32, (nb, 1), 0) * BLK
    cum_b = jnp.broadcast_to(cum, (nb, E))
    total = jnp.max(cum_b, axis=1, keepdims=True)
    s_cl = jnp.minimum(sb.astype(jnp.float32), total - 1.0)
    e_b = jnp.sum(jnp.where(cum_b <= s_cl, 1, 0), axis=1, keepdims=True)
    blke_ref[...] = e_b.astype(jnp.int32)
    blkv_ref[...] = (sb.astype(jnp.float32) < total).astype(jnp.int32)


def _router_call(x, wr):
    return pl.pallas_call(
        _router_body,
        out_shape=[
            jax.ShapeDtypeStruct((T, K), jnp.int32),
            jax.ShapeDtypeStruct((T, K), jnp.float32),
            jax.ShapeDtypeStruct((NBLK, 1), jnp.int32),
            jax.ShapeDtypeStruct((NBLK, 1), jnp.int32),
        ],
    )(x, wr)


@functools.lru_cache(maxsize=None)
def _sc_mesh():
    return plsc.VectorSubcoreMesh(core_axis_name="c", subcore_axis_name="s")
_EPT = ENTRIES // 16
_EROWS = _EPT // 128
_SEG = CAP // 16


def _build_body(pos_hbm, g_hbm, x_hbm, xs_hbm, gs_hbm,
                sp_ids, sp_g, posb, gb, vals, zi, zf, idxb, rowsb, sem):
    c = lax.axis_index("c")
    s = lax.axis_index("s")
    wid = c * 16 + s

    for j in range(_SEG // 16):
        zi[pl.ds(16 * j, 16)] = jnp.zeros((16,), jnp.int32)
        zf[pl.ds(16 * j, 16)] = jnp.zeros((16,), jnp.float32)
    pltpu.sync_copy(zi, sp_ids.at[pl.ds(s * _SEG, _SEG)])
    pltpu.sync_copy(zf, sp_g.at[pl.ds(s * _SEG, _SEG)])
    plsc.subcore_barrier()

    pltpu.sync_copy(pos_hbm.at[pl.ds(s * _EROWS, _EROWS)], posb)
    pltpu.sync_copy(g_hbm.at[pl.ds(s * _EROWS, _EROWS)], gb)
    for r in range(_EROWS):
        for q in range(8):
            base = s * _EPT + r * 128 + q * 16
            ent = base + lax.iota(jnp.int32, 16)
            vals[r, pl.ds(q * 16, 16)] = lax.shift_right_logical(ent, 1)
    for r in range(_EROWS):
        pltpu.sync_copy(vals.at[r], sp_ids.at[posb.at[r]])
        pltpu.sync_copy(gb.at[r], sp_g.at[posb.at[r]])
    plsc.subcore_barrier()

    for k in range(ROWS_PER_TILE // GCHUNK):
        row0 = wid * ROWS_PER_TILE + k * GCHUNK
        pltpu.sync_copy(sp_ids.at[pl.ds(row0, GCHUNK)], idxb)
        pltpu.async_copy(x_hbm.at[idxb], rowsb, sem).wait()
        pltpu.sync_copy(rowsb, xs_hbm.at[pl.ds(row0, GCHUNK)])

    @pl.when(c == 0)
    def _():
        pltpu.sync_copy(sp_g.at[pl.ds(s * _SEG, _SEG)],
                        gs_hbm.at[pl.ds(s * _SEG, _SEG)])


def _build_call(pos64, g64, x):
    return pl.kernel(
        _build_body,
        out_type=[
            jax.ShapeDtypeStruct((CAP, D), jnp.float32),
            jax.ShapeDtypeStruct((CAP,), jnp.float32),
        ],
        mesh=_sc_mesh(),
        scratch_types=[
            pltpu.VMEM_SHARED((CAP,), jnp.int32),
            pltpu.VMEM_SHARED((CAP,), jnp.float32),
            pltpu.VMEM((_EROWS, 128), jnp.int32),
            pltpu.VMEM((_EROWS, 128), jnp.float32),
            pltpu.VMEM((_EROWS, 128), jnp.int32),
            pltpu.VMEM((_SEG,), jnp.int32),
            pltpu.VMEM((_SEG,), jnp.float32),
            pltpu.VMEM((GCHUNK,), jnp.int32),
            pltpu.VMEM((GCHUNK, D), jnp.float32),
            pltpu.SemaphoreType.DMA,
        ],
    )(pos64, g64, x)


def _ffn_body(blke_ref, blkv_ref, xs_ref, wg_ref, wu_ref, wd_ref, gs_ref,
              ys_ref):
    i = pl.program_id(0)

    @pl.when(blkv_ref[i] == 1)
    def _():
        xb = xs_ref[...]
        g = lax.dot_general(xb, wg_ref[0], (((1,), (1,)), ((), ())),
                            preferred_element_type=jnp.float32)
        u = lax.dot_general(xb, wu_ref[0], (((1,), (1,)), ((), ())),
                            preferred_element_type=jnp.float32)
        h = (g / (1.0 + jnp.exp(-g))) * u
        h = h * gs_ref[...]
        ys_ref[...] = lax.dot_general(h, wd_ref[0], (((1,), (1,)), ((), ())),
                                      preferred_element_type=jnp.float32)


def _ffn_call(blke, blkv, xs, wg, wu, wd, gs):
    grid_spec = pltpu.PrefetchScalarGridSpec(
        num_scalar_prefetch=2,
        grid=(NBLK,),
        in_specs=[
            pl.BlockSpec((BLK, D), lambda i, be, bv: (i, 0)),
            pl.BlockSpec((1, F, D), lambda i, be, bv: (be[i], 0, 0)),
            pl.BlockSpec((1, F, D), lambda i, be, bv: (be[i], 0, 0)),
            pl.BlockSpec((1, D, F), lambda i, be, bv: (be[i], 0, 0)),
            pl.BlockSpec((BLK, 1), lambda i, be, bv: (i, 0)),
        ],
        out_specs=pl.BlockSpec((BLK, D), lambda i, be, bv: (i, 0)),
    )
    return pl.pallas_call(
        _ffn_body,
        grid_spec=grid_spec,
        out_shape=jax.ShapeDtypeStruct((CAP, D), jnp.float32),
        compiler_params=pltpu.CompilerParams(
            dimension_semantics=("arbitrary",)),
    )(blke, blkv, xs, wg, wu, wd, gs)


_TPT = T // NTILES
_CCH = 32
_NSL = _CCH * D // 16


def _combine_body(x_hbm, ys_hbm, p0_hbm, p1_hbm, out_hbm,
                  xbuf, y0, y1, idxb, sem):
    c = lax.axis_index("c")
    s = lax.axis_index("s")
    wid = c * 16 + s
    for ch in range(_TPT // _CCH):
        t0 = wid * _TPT + ch * _CCH
        pltpu.sync_copy(x_hbm.at[pl.ds(t0, _CCH)], xbuf)
        pltpu.sync_copy(p0_hbm.at[pl.ds(t0, _CCH)], idxb)
        pltpu.async_copy(ys_hbm.at[idxb], y0, sem).wait()
        pltpu.sync_copy(p1_hbm.at[pl.ds(t0, _CCH)], idxb)
        pltpu.async_copy(ys_hbm.at[idxb], y1, sem).wait()

        def add_body(i, carry):
            r = i // (D // 16)
            c0 = (i % (D // 16)) * 16
            sl = pl.ds(c0, 16)
            xbuf[r, sl] = xbuf[r, sl] + y0[r, sl] + y1[r, sl]
            return carry

        lax.fori_loop(0, _NSL, add_body, 0)
        pltpu.sync_copy(xbuf, out_hbm.at[pl.ds(t0, _CCH)])


def _combine_call(x, ys, p0, p1):
    return pl.kernel(
        _combine_body,
        out_type=jax.ShapeDtypeStruct((T, D), jnp.float32),
        mesh=_sc_mesh(),
        scratch_types=[
            pltpu.VMEM((_CCH, D), jnp.float32),
            pltpu.VMEM((_CCH, D), jnp.float32),
            pltpu.VMEM((_CCH, D), jnp.float32),
            pltpu.VMEM((_CCH,), jnp.int32),
            pltpu.SemaphoreType.DMA,
        ],
    )(x, ys, p0, p1)


def kernel(x, Wr, Wg, Wu, Wd):
    pos, gates, blke, blkv = _router_call(x, Wr)
    pos64 = pos.reshape(ENTRIES // 128, 128)
    g64 = gates.reshape(ENTRIES // 128, 128)
    xs, gs = _build_call(pos64, g64, x)
    ys = _ffn_call(blke.reshape(NBLK), blkv.reshape(NBLK),
                   xs, Wg, Wu, Wd, gs.reshape(CAP, 1))
    p0 = pos[:, 0].reshape(T)
    p1 = pos[:, 1].reshape(T)
    return _combine_call(x, ys, p0, p1)


# speedup vs baseline: 1.1017x; 1.0739x over previous
import functools

import jax
import jax.numpy as jnp
from jax import lax
from jax.experimental import pallas as pl
from jax.experimental.pallas import tpu as pltpu
from jax.experimental.pallas import tpu_sc as plsc

T = 4096
D = 1024
F = 512
E = 16
K = 2
BLK = 256
CAP = T * K + E * BLK
NBLK = CAP // BLK
NTILES = 32
ROWS_PER_TILE = CAP // NTILES
GCHUNK = 32
ENTRIES = T * K


def _router_body(x_ref, wr_ref, pos_ref, gate_ref, blke_ref, blkv_ref):
    x = x_ref[...]
    wr = wr_ref[...]
    logits = lax.dot_general(x, wr, (((1,), (1,)), ((), ())),
                             preferred_element_type=jnp.float32,
                             precision=lax.Precision.HIGHEST)
    m = jnp.max(logits, axis=1, keepdims=True)
    p = jnp.exp(logits - m)
    p = p / jnp.sum(p, axis=1, keepdims=True)

    lane = lax.broadcasted_iota(jnp.int32, (T, E), 1)
    m1 = jnp.max(p, axis=1, keepdims=True)
    i1 = jnp.min(jnp.where(p == m1, lane, E), axis=1, keepdims=True)
    p2 = jnp.where(lane == i1, -1.0, p)
    m2 = jnp.max(p2, axis=1, keepdims=True)
    i2 = jnp.min(jnp.where(p2 == m2, lane, E), axis=1, keepdims=True)
    s = m1 + m2
    g1 = m1 / s
    g2 = m2 / s

    oh1 = (lane == i1)
    oh2 = (lane == i2)
    ohc = jnp.where(oh1 | oh2, 1.0, 0.0)

    c = ohc
    k = 1
    while k < T:
        shifted = jnp.concatenate(
            [jnp.zeros((k, E), jnp.float32), c[: T - k, :]], axis=0)
        c = c + shifted
        k *= 2
    c_excl = c - ohc
    r1 = jnp.sum(jnp.where(oh1, c_excl, 0.0), axis=1, keepdims=True)
    r2 = jnp.sum(jnp.where(oh2, c_excl, 0.0), axis=1, keepdims=True)

    counts = c[T - 1 : T, :]
    padded = jnp.floor((counts + (BLK - 1)) / BLK) * BLK
    ri = lax.broadcasted_iota(jnp.int32, (E, E), 0)
    ci = lax.broadcasted_iota(jnp.int32, (E, E), 1)
    ut = jnp.where(ri <= ci, 1.0, 0.0)
    cum = lax.dot_general(padded, ut, (((1,), (0,)), ((), ())),
                          preferred_element_type=jnp.float32)
    offs = cum - padded

    offs_b = jnp.broadcast_to(offs, (T, E))
    o1 = jnp.sum(jnp.where(oh1, offs_b, 0.0), axis=1, keepdims=True)
    o2 = jnp.sum(jnp.where(oh2, offs_b, 0.0), axis=1, keepdims=True)
    pos1 = (o1 + r1).astype(jnp.int32)
    pos2 = (o2 + r2).astype(jnp.int32)
    pos_ref[...] = jnp.concatenate([pos1, pos2], axis=1)
    gate_ref[...] = jnp.concatenate([g1, g2], axis=1)

    nb = blke_ref.shape[0]
    sb = lax.broadcasted_iota(jnp.int32, (nb, 1), 0) * BLK
    cum_b = jnp.broadcast_to(cum, (nb, E))
    total = jnp.max(cum_b, axis=1, keepdims=True)
    s_cl = jnp.minimum(sb.astype(jnp.float32), total - 1.0)
    e_b = jnp.sum(jnp.where(cum_b <= s_cl, 1, 0), axis=1, keepdims=True)
    blke_ref[...] = e_b.astype(jnp.int32)
    blkv_ref[...] = (sb.astype(jnp.float32) < total).astype(jnp.int32)


def _router_call(x, wr):
    return pl.pallas_call(
        _router_body,
        out_shape=[
            jax.ShapeDtypeStruct((T, K), jnp.int32),
            jax.ShapeDtypeStruct((T, K), jnp.float32),
            jax.ShapeDtypeStruct((NBLK, 1), jnp.int32),
            jax.ShapeDtypeStruct((NBLK, 1), jnp.int32),
        ],
    )(x, wr)


@functools.lru_cache(maxsize=None)
def _sc_mesh():
    return plsc.VectorSubcoreMesh(core_axis_name="c", subcore_axis_name="s")
_EPT = ENTRIES // 16
_EROWS = _EPT // 128
_SEG = CAP // 16


def _build_body(pos_hbm, g_hbm, x_hbm, xs_hbm, gs_hbm,
                sp_ids, sp_g, posb, gb, vals, zi, zf, idxb, rowsb,
                gsem, wsem):
    c = lax.axis_index("c")
    s = lax.axis_index("s")
    wid = c * 16 + s

    for j in range(_SEG // 16):
        zi[pl.ds(16 * j, 16)] = jnp.zeros((16,), jnp.int32)
        zf[pl.ds(16 * j, 16)] = jnp.zeros((16,), jnp.float32)
    pltpu.sync_copy(zi, sp_ids.at[pl.ds(s * _SEG, _SEG)])
    pltpu.sync_copy(zf, sp_g.at[pl.ds(s * _SEG, _SEG)])
    plsc.subcore_barrier()

    pltpu.sync_copy(pos_hbm.at[pl.ds(s * _EROWS, _EROWS)], posb)
    pltpu.sync_copy(g_hbm.at[pl.ds(s * _EROWS, _EROWS)], gb)
    for r in range(_EROWS):
        for q in range(8):
            base = s * _EPT + r * 128 + q * 16
            ent = base + lax.iota(jnp.int32, 16)
            vals[r, pl.ds(q * 16, 16)] = lax.shift_right_logical(ent, 1)
    for r in range(_EROWS):
        pltpu.sync_copy(vals.at[r], sp_ids.at[posb.at[r]])
        pltpu.sync_copy(gb.at[r], sp_g.at[posb.at[r]])
    plsc.subcore_barrier()

    nch = ROWS_PER_TILE // GCHUNK

    def _start_gather(k, b):
        row0 = wid * ROWS_PER_TILE + k * GCHUNK
        pltpu.sync_copy(sp_ids.at[pl.ds(row0, GCHUNK)], idxb[b])
        return pltpu.async_copy(x_hbm.at[idxb[b]], rowsb[b], gsem[b])

    gd = [_start_gather(b, b) for b in range(3)]
    for k in range(nch):
        b = k % 3
        row0 = wid * ROWS_PER_TILE + k * GCHUNK
        gd[b].wait()
        wd = pltpu.async_copy(rowsb[b], xs_hbm.at[pl.ds(row0, GCHUNK)],
                              wsem[b])
        if k + 3 < nch:
            wd.wait()
            gd[b] = _start_gather(k + 3, b)
        else:
            wd.wait()

    @pl.when(c == 0)
    def _():
        pltpu.sync_copy(sp_g.at[pl.ds(s * _SEG, _SEG)],
                        gs_hbm.at[pl.ds(s * _SEG, _SEG)])


def _build_call(pos64, g64, x):
    return pl.kernel(
        _build_body,
        out_type=[
            jax.ShapeDtypeStruct((CAP, D), jnp.float32),
            jax.ShapeDtypeStruct((CAP,), jnp.float32),
        ],
        mesh=_sc_mesh(),
        scratch_types=[
            pltpu.VMEM_SHARED((CAP,), jnp.int32),
            pltpu.VMEM_SHARED((CAP,), jnp.float32),
            pltpu.VMEM((_EROWS, 128), jnp.int32),
            pltpu.VMEM((_EROWS, 128), jnp.float32),
            pltpu.VMEM((_EROWS, 128), jnp.int32),
            pltpu.VMEM((_SEG,), jnp.int32),
            pltpu.VMEM((_SEG,), jnp.float32),
            [pltpu.VMEM((GCHUNK,), jnp.int32) for _ in range(3)],
            [pltpu.VMEM((GCHUNK, D), jnp.float32) for _ in range(3)],
            [pltpu.SemaphoreType.DMA for _ in range(3)],
            [pltpu.SemaphoreType.DMA for _ in range(3)],
        ],
    )(pos64, g64, x)


def _ffn_body(blke_ref, blkv_ref, xs_ref, wg_ref, wu_ref, wd_ref, gs_ref,
              ys_ref):
    i = pl.program_id(0)

    @pl.when(blkv_ref[i] == 1)
    def _():
        xb = xs_ref[...]
        g = lax.dot_general(xb, wg_ref[0], (((1,), (1,)), ((), ())),
                            preferred_element_type=jnp.float32)
        u = lax.dot_general(xb, wu_ref[0], (((1,), (1,)), ((), ())),
                            preferred_element_type=jnp.float32)
        h = (g / (1.0 + jnp.exp(-g))) * u
        h = h * gs_ref[...]
        ys_ref[...] = lax.dot_general(h, wd_ref[0], (((1,), (1,)), ((), ())),
                                      preferred_element_type=jnp.float32)


def _ffn_call(blke, blkv, xs, wg, wu, wd, gs):
    grid_spec = pltpu.PrefetchScalarGridSpec(
        num_scalar_prefetch=2,
        grid=(NBLK,),
        in_specs=[
            pl.BlockSpec((BLK, D), lambda i, be, bv: (i, 0)),
            pl.BlockSpec((1, F, D), lambda i, be, bv: (be[i], 0, 0)),
            pl.BlockSpec((1, F, D), lambda i, be, bv: (be[i], 0, 0)),
            pl.BlockSpec((1, D, F), lambda i, be, bv: (be[i], 0, 0)),
            pl.BlockSpec((BLK, 1), lambda i, be, bv: (i, 0)),
        ],
        out_specs=pl.BlockSpec((BLK, D), lambda i, be, bv: (i, 0)),
    )
    return pl.pallas_call(
        _ffn_body,
        grid_spec=grid_spec,
        out_shape=jax.ShapeDtypeStruct((CAP, D), jnp.float32),
        compiler_params=pltpu.CompilerParams(
            dimension_semantics=("arbitrary",)),
    )(blke, blkv, xs, wg, wu, wd, gs)


_TPT = T // NTILES
_CCH = 16
_NCCH = _TPT // _CCH


def _combine_body(x_hbm, ys_hbm, p0_hbm, p1_hbm, out_hbm,
                  xbuf, y0, y1, idx0, idx1, xsem, g0sem, g1sem, wsem):
    c = lax.axis_index("c")
    s = lax.axis_index("s")
    wid = c * 16 + s

    def _start_in(k, b):
        t0 = wid * _TPT + k * _CCH
        xd = pltpu.async_copy(x_hbm.at[pl.ds(t0, _CCH)], xbuf[b], xsem[b])
        pltpu.sync_copy(p0_hbm.at[pl.ds(t0, _CCH)], idx0[b])
        pltpu.sync_copy(p1_hbm.at[pl.ds(t0, _CCH)], idx1[b])
        g0 = pltpu.async_copy(ys_hbm.at[idx0[b]], y0[b], g0sem[b])
        g1 = pltpu.async_copy(ys_hbm.at[idx1[b]], y1[b], g1sem[b])
        return xd, g0, g1

    ind = [_start_in(b, b) for b in range(2)]
    for k in range(_NCCH):
        b = k % 2
        t0 = wid * _TPT + k * _CCH
        for d in ind[b]:
            d.wait()

        def add_body(r, carry):
            for q in range(D // 16):
                sl = pl.ds(q * 16, 16)
                xbuf[b][r, sl] = xbuf[b][r, sl] + y0[b][r, sl] + y1[b][r, sl]
            return carry

        lax.fori_loop(0, _CCH, add_body, 0)
        wd = pltpu.async_copy(xbuf[b], out_hbm.at[pl.ds(t0, _CCH)], wsem[b])
        if k + 2 < _NCCH:
            wd.wait()
            ind[b] = _start_in(k + 2, b)
        else:
            wd.wait()


def _combine_call(x, ys, p0, p1):
    return pl.kernel(
        _combine_body,
        out_type=jax.ShapeDtypeStruct((T, D), jnp.float32),
        mesh=_sc_mesh(),
        scratch_types=[
            [pltpu.VMEM((_CCH, D), jnp.float32) for _ in range(2)],
            [pltpu.VMEM((_CCH, D), jnp.float32) for _ in range(2)],
            [pltpu.VMEM((_CCH, D), jnp.float32) for _ in range(2)],
            [pltpu.VMEM((_CCH,), jnp.int32) for _ in range(2)],
            [pltpu.VMEM((_CCH,), jnp.int32) for _ in range(2)],
            [pltpu.SemaphoreType.DMA for _ in range(2)],
            [pltpu.SemaphoreType.DMA for _ in range(2)],
            [pltpu.SemaphoreType.DMA for _ in range(2)],
            [pltpu.SemaphoreType.DMA for _ in range(2)],
        ],
    )(x, ys, p0, p1)


def kernel(x, Wr, Wg, Wu, Wd):
    pos, gates, blke, blkv = _router_call(x, Wr)
    pos64 = pos.reshape(ENTRIES // 128, 128)
    g64 = gates.reshape(ENTRIES // 128, 128)
    xs, gs = _build_call(pos64, g64, x)
    ys = _ffn_call(blke.reshape(NBLK), blkv.reshape(NBLK),
                   xs, Wg, Wu, Wd, gs.reshape(CAP, 1))
    p0 = pos[:, 0].reshape(T)
    p1 = pos[:, 1].reshape(T)
    return _combine_call(x, ys, p0, p1)


# speedup vs baseline: 2.2185x; 2.0138x over previous
import functools

import jax
import jax.numpy as jnp
from jax import lax
from jax.experimental import pallas as pl
from jax.experimental.pallas import tpu as pltpu
from jax.experimental.pallas import tpu_sc as plsc

T = 4096
D = 1024
F = 512
E = 16
K = 2
BLK = 256
CAP = T * K + E * BLK
NBLK = CAP // BLK
NTILES = 32
ROWS_PER_TILE = CAP // NTILES
GCHUNK = 24
ENTRIES = T * K


def _router_body(x_ref, wr_ref, pos_ref, gate_ref, blke_ref, blkv_ref,
                 xp_ref):
    x = x_ref[...]
    wr = wr_ref[...]
    lo = lax.bitcast_convert_type(x[:, : D // 2].astype(jnp.bfloat16),
                                  jnp.uint16).astype(jnp.uint32)
    hi = lax.bitcast_convert_type(x[:, D // 2 :].astype(jnp.bfloat16),
                                  jnp.uint16).astype(jnp.uint32)
    xp_ref[...] = lo | (hi << 16)
    logits = lax.dot_general(x, wr, (((1,), (1,)), ((), ())),
                             preferred_element_type=jnp.float32,
                             precision=lax.Precision.HIGHEST)
    m = jnp.max(logits, axis=1, keepdims=True)
    p = jnp.exp(logits - m)
    p = p / jnp.sum(p, axis=1, keepdims=True)

    lane = lax.broadcasted_iota(jnp.int32, (T, E), 1)
    m1 = jnp.max(p, axis=1, keepdims=True)
    i1 = jnp.min(jnp.where(p == m1, lane, E), axis=1, keepdims=True)
    p2 = jnp.where(lane == i1, -1.0, p)
    m2 = jnp.max(p2, axis=1, keepdims=True)
    i2 = jnp.min(jnp.where(p2 == m2, lane, E), axis=1, keepdims=True)
    s = m1 + m2
    g1 = m1 / s
    g2 = m2 / s

    oh1 = (lane == i1)
    oh2 = (lane == i2)
    ohc = jnp.where(oh1 | oh2, 1.0, 0.0)

    c = ohc
    k = 1
    while k < T:
        shifted = jnp.concatenate(
            [jnp.zeros((k, E), jnp.float32), c[: T - k, :]], axis=0)
        c = c + shifted
        k *= 2
    c_excl = c - ohc
    r1 = jnp.sum(jnp.where(oh1, c_excl, 0.0), axis=1, keepdims=True)
    r2 = jnp.sum(jnp.where(oh2, c_excl, 0.0), axis=1, keepdims=True)

    counts = c[T - 1 : T, :]
    padded = jnp.floor((counts + (BLK - 1)) / BLK) * BLK
    ri = lax.broadcasted_iota(jnp.int32, (E, E), 0)
    ci = lax.broadcasted_iota(jnp.int32, (E, E), 1)
    ut = jnp.where(ri <= ci, 1.0, 0.0)
    cum = lax.dot_general(padded, ut, (((1,), (0,)), ((), ())),
                          preferred_element_type=jnp.float32)
    offs = cum - padded

    offs_b = jnp.broadcast_to(offs, (T, E))
    o1 = jnp.sum(jnp.where(oh1, offs_b, 0.0), axis=1, keepdims=True)
    o2 = jnp.sum(jnp.where(oh2, offs_b, 0.0), axis=1, keepdims=True)
    pos1 = (o1 + r1).astype(jnp.int32)
    pos2 = (o2 + r2).astype(jnp.int32)
    pos_ref[...] = jnp.concatenate([pos1, pos2], axis=1)
    gate_ref[...] = jnp.concatenate([g1, g2], axis=1)

    nb = blke_ref.shape[0]
    sb = lax.broadcasted_iota(jnp.int32, (nb, 1), 0) * BLK
    cum_b = jnp.broadcast_to(cum, (nb, E))
    total = jnp.max(cum_b, axis=1, keepdims=True)
    s_cl = jnp.minimum(sb.astype(jnp.float32), total - 1.0)
    e_b = jnp.sum(jnp.where(cum_b <= s_cl, 1, 0), axis=1, keepdims=True)
    blke_ref[...] = e_b.astype(jnp.int32)
    blkv_ref[...] = (sb.astype(jnp.float32) < total).astype(jnp.int32)


def _router_call(x, wr):
    return pl.pallas_call(
        _router_body,
        out_shape=[
            jax.ShapeDtypeStruct((T, K), jnp.int32),
            jax.ShapeDtypeStruct((T, K), jnp.float32),
            jax.ShapeDtypeStruct((NBLK, 1), jnp.int32),
            jax.ShapeDtypeStruct((NBLK, 1), jnp.int32),
            jax.ShapeDtypeStruct((T, D // 2), jnp.uint32),
        ],
        compiler_params=pltpu.CompilerParams(
            vmem_limit_bytes=100 * 1024 * 1024),
    )(x, wr)


@functools.lru_cache(maxsize=None)
def _sc_mesh():
    return plsc.VectorSubcoreMesh(core_axis_name="c", subcore_axis_name="s")
_EPT = ENTRIES // 16
_EROWS = _EPT // 128
_SEG = CAP // 16


_XPT = T // NTILES
_XCH = 32


def _build_body(pos_hbm, g_hbm, p0_hbm, p1_hbm, xh_hbm, xs_hbm, gs_hbm,
                sp_g, posb, gb, zf, rowsb, idx0, idx1, rsem, w0sem, w1sem):
    c = lax.axis_index("c")
    s = lax.axis_index("s")
    wid = c * 16 + s

    @pl.when(c == 0)
    def _():
        with jax.named_scope("ph_ab"):
            for j in range(_SEG // 16):
                zf[pl.ds(16 * j, 16)] = jnp.zeros((16,), jnp.float32)
            pltpu.sync_copy(zf, sp_g.at[pl.ds(s * _SEG, _SEG)])
            plsc.subcore_barrier()
            pltpu.sync_copy(pos_hbm.at[pl.ds(s * _EROWS, _EROWS)], posb)
            pltpu.sync_copy(g_hbm.at[pl.ds(s * _EROWS, _EROWS)], gb)
            for r in range(_EROWS):
                pltpu.sync_copy(gb.at[r], sp_g.at[posb.at[r]])
            plsc.subcore_barrier()
            pltpu.sync_copy(sp_g.at[pl.ds(s * _SEG, _SEG)],
                            gs_hbm.at[pl.ds(s * _SEG, _SEG)])

    nch = _XPT // _XCH
    nbuf = 3

    def _start_chunk(k, b):
        t0 = wid * _XPT + k * _XCH
        rd = pltpu.async_copy(xh_hbm.at[pl.ds(t0, _XCH)], rowsb[b], rsem[b])
        pltpu.sync_copy(p0_hbm.at[pl.ds(t0, _XCH)], idx0[b])
        pltpu.sync_copy(p1_hbm.at[pl.ds(t0, _XCH)], idx1[b])
        return rd

    with jax.named_scope("ph_c"):
        rd = [None] * nbuf
        wd = [None] * nbuf
        for k in range(min(nbuf, nch)):
            rd[k % nbuf] = _start_chunk(k, k % nbuf)
        for k in range(nch):
            b = k % nbuf
            rd[b].wait()
            w0 = pltpu.async_copy(rowsb[b], xs_hbm.at[idx0[b]], w0sem[b])
            w1 = pltpu.async_copy(rowsb[b], xs_hbm.at[idx1[b]], w1sem[b])
            wd[b] = (w0, w1)
            nk = k + nbuf
            if nk < nch:
                w0.wait()
                w1.wait()
                wd[b] = None
                rd[b] = _start_chunk(nk, b)
        for b in range(nbuf):
            if wd[b] is not None:
                wd[b][0].wait()
                wd[b][1].wait()


def _build_call(pos64, g64, p0, p1, xh):
    return pl.kernel(
        _build_body,
        out_type=[
            jax.ShapeDtypeStruct((CAP, D // 2), jnp.uint32),
            jax.ShapeDtypeStruct((CAP,), jnp.float32),
        ],
        mesh=_sc_mesh(),
        scratch_types=[
            pltpu.VMEM_SHARED((CAP,), jnp.float32),
            pltpu.VMEM((_EROWS, 128), jnp.int32),
            pltpu.VMEM((_EROWS, 128), jnp.float32),
            pltpu.VMEM((_SEG,), jnp.float32),
            [pltpu.VMEM((_XCH, D // 2), jnp.uint32) for _ in range(3)],
            [pltpu.VMEM((_XCH,), jnp.int32) for _ in range(3)],
            [pltpu.VMEM((_XCH,), jnp.int32) for _ in range(3)],
            [pltpu.SemaphoreType.DMA for _ in range(3)],
            [pltpu.SemaphoreType.DMA for _ in range(3)],
            [pltpu.SemaphoreType.DMA for _ in range(3)],
        ],
    )(pos64, g64, p0, p1, xh)


def _ffn_body(blke_ref, blkv_ref, xs_ref, wg_ref, wu_ref, wd_ref, gs_ref,
              ys_ref):
    i = pl.program_id(0)

    @pl.when(blkv_ref[i] == 1)
    def _():
        xu = xs_ref[...]
        xlo = lax.bitcast_convert_type(xu << 16, jnp.float32)
        xhi = lax.bitcast_convert_type(xu & jnp.uint32(0xFFFF0000),
                                       jnp.float32)
        hd = D // 2

        def mm(a_lo, a_hi, w):
            return (lax.dot_general(a_lo, w[:, :hd], (((1,), (1,)), ((), ())),
                                    preferred_element_type=jnp.float32)
                    + lax.dot_general(a_hi, w[:, hd:], (((1,), (1,)), ((), ())),
                                      preferred_element_type=jnp.float32))

        g = mm(xlo, xhi, wg_ref[0])
        u = mm(xlo, xhi, wu_ref[0])
        h = (g / (1.0 + jnp.exp(-g))) * u
        h = h * gs_ref[...]
        ys_ref[...] = lax.dot_general(h, wd_ref[0], (((1,), (1,)), ((), ())),
                                      preferred_element_type=jnp.float32)


def _ffn_call(blke, blkv, xs, wg, wu, wd, gs):
    grid_spec = pltpu.PrefetchScalarGridSpec(
        num_scalar_prefetch=2,
        grid=(NBLK,),
        in_specs=[
            pl.BlockSpec((BLK, D // 2), lambda i, be, bv: (i, 0)),
            pl.BlockSpec((1, F, D), lambda i, be, bv: (be[i], 0, 0)),
            pl.BlockSpec((1, F, D), lambda i, be, bv: (be[i], 0, 0)),
            pl.BlockSpec((1, D, F), lambda i, be, bv: (be[i], 0, 0)),
            pl.BlockSpec((BLK, 1), lambda i, be, bv: (i, 0)),
        ],
        out_specs=pl.BlockSpec((BLK, D), lambda i, be, bv: (i, 0)),
    )
    return pl.pallas_call(
        _ffn_body,
        grid_spec=grid_spec,
        out_shape=jax.ShapeDtypeStruct((CAP, D), jnp.float32),
        compiler_params=pltpu.CompilerParams(
            dimension_semantics=("arbitrary",)),
    )(blke, blkv, xs, wg, wu, wd, gs)


_TPT = T // NTILES
_CCH = 16
_NCCH = _TPT // _CCH


def _combine_body(x_hbm, ys_hbm, p0_hbm, p1_hbm, out_hbm,
                  xbuf, y0, y1, idx0, idx1, xsem, g0sem, g1sem, wsem):
    c = lax.axis_index("c")
    s = lax.axis_index("s")
    wid = c * 16 + s

    def _start_in(k, b):
        t0 = wid * _TPT + k * _CCH
        xd = pltpu.async_copy(x_hbm.at[pl.ds(t0, _CCH)], xbuf[b], xsem[b])
        pltpu.sync_copy(p0_hbm.at[pl.ds(t0, _CCH)], idx0[b])
        pltpu.sync_copy(p1_hbm.at[pl.ds(t0, _CCH)], idx1[b])
        g0 = pltpu.async_copy(ys_hbm.at[idx0[b]], y0[b], g0sem[b])
        g1 = pltpu.async_copy(ys_hbm.at[idx1[b]], y1[b], g1sem[b])
        return xd, g0, g1

    ind = [_start_in(b, b) for b in range(2)]
    for k in range(_NCCH):
        b = k % 2
        t0 = wid * _TPT + k * _CCH
        for d in ind[b]:
            d.wait()

        def add_body(r, carry):
            for q in range(D // 16):
                sl = pl.ds(q * 16, 16)
                xbuf[b][r, sl] = xbuf[b][r, sl] + y0[b][r, sl] + y1[b][r, sl]
            return carry

        lax.fori_loop(0, _CCH, add_body, 0)
        wd = pltpu.async_copy(xbuf[b], out_hbm.at[pl.ds(t0, _CCH)], wsem[b])
        if k + 2 < _NCCH:
            wd.wait()
            ind[b] = _start_in(k + 2, b)
        else:
            wd.wait()


def _combine_call(x, ys, p0, p1):
    return pl.kernel(
        _combine_body,
        out_type=jax.ShapeDtypeStruct((T, D), jnp.float32),
        mesh=_sc_mesh(),
        scratch_types=[
            [pltpu.VMEM((_CCH, D), jnp.float32) for _ in range(2)],
            [pltpu.VMEM((_CCH, D), jnp.float32) for _ in range(2)],
            [pltpu.VMEM((_CCH, D), jnp.float32) for _ in range(2)],
            [pltpu.VMEM((_CCH,), jnp.int32) for _ in range(2)],
            [pltpu.VMEM((_CCH,), jnp.int32) for _ in range(2)],
            [pltpu.SemaphoreType.DMA for _ in range(2)],
            [pltpu.SemaphoreType.DMA for _ in range(2)],
            [pltpu.SemaphoreType.DMA for _ in range(2)],
            [pltpu.SemaphoreType.DMA for _ in range(2)],
        ],
    )(x, ys, p0, p1)


def kernel(x, Wr, Wg, Wu, Wd):
    pos, gates, blke, blkv, xp = _router_call(x, Wr)
    pos64 = pos.reshape(ENTRIES // 128, 128)
    g64 = gates.reshape(ENTRIES // 128, 128)
    p0 = pos[:, 0].reshape(T)
    p1 = pos[:, 1].reshape(T)
    xs, gs = _build_call(pos64, g64, p0, p1, xp)
    ys = _ffn_call(blke.reshape(NBLK), blkv.reshape(NBLK),
                   xs, Wg, Wu, Wd, gs.reshape(CAP, 1))
    return _combine_call(x, ys, p0, p1)
